# Initial kernel scaffold; baseline (speedup 1.0000x reference)
import jax
import jax.numpy as jnp
from jax.experimental import pallas as pl


def kernel(x, edge_index, W1, b1, W2, b2, W3, b3):
    raise NotImplementedError("write your pallas kernel here")


# speedup vs baseline: 21.2325x; 21.2325x over previous
import functools

import jax
import jax.numpy as jnp
from jax import lax
from jax.experimental import pallas as pl
from jax.experimental.pallas import tpu as pltpu
from jax.experimental.pallas import tpu_sc as plsc

N = 10000
E = 320000
D_IN = 128
D_H = 128
D_OUT = 64
DH2 = D_H // 2

NC = 2
NS = 16
NW = NC * NS
K = 100
NB = E // (NW * K)
RPT = 632
DEGW = 16

_MESH = plsc.VectorSubcoreMesh(
    core_axis_name="c", subcore_axis_name="s", num_cores=NC, num_subcores=NS
)


def _make_deg_kernel():
    @functools.partial(
        pl.kernel,
        out_type=jax.ShapeDtypeStruct((NC, N, DEGW), jnp.float32),
        mesh=_MESH,
        scratch_types=[
            pltpu.VMEM((NB, K), jnp.int32),
            pltpu.VMEM((K, DEGW), jnp.float32),
            pltpu.VMEM_SHARED((N, DEGW), jnp.float32),
        ],
        compiler_params=pltpu.CompilerParams(use_tc_tiling_on_sc=False),
    )
    def deg_kernel(col3, zblk, out, col_v, ones_v, acc):
        c = lax.axis_index("c")
        s = lax.axis_index("s")
        w = s * NC + c
        off = pl.multiple_of(jnp.minimum(s * RPT, N - RPT), 8)
        pltpu.sync_copy(col3.at[w], col_v)

        def fill(r, carry):
            ones_v[r, :] = jnp.ones((DEGW,), jnp.float32)
            return carry

        lax.fori_loop(0, K, fill, 0)
        pltpu.sync_copy(zblk, acc.at[pl.ds(off, RPT)])
        plsc.subcore_barrier()

        def body(j, carry):
            pltpu.sync_copy(ones_v, acc.at[col_v.at[j]], add=True)
            return carry

        lax.fori_loop(0, NB, body, 0)
        plsc.subcore_barrier()
        pltpu.sync_copy(acc.at[pl.ds(off, RPT)], out.at[c, pl.ds(off, RPT)])

    return deg_kernel


def _make_agg_kernel(D):
    @functools.partial(
        pl.kernel,
        out_type=jax.ShapeDtypeStruct((NC, N, D), jnp.float32),
        mesh=_MESH,
        scratch_types=[
            pltpu.VMEM((NB, K), jnp.int32),
            pltpu.VMEM((NB, K), jnp.int32),
            pltpu.VMEM((K, D), jnp.float32),
            pltpu.VMEM((K, D), jnp.float32),
            pltpu.VMEM_SHARED((N, D), jnp.float32),
            pltpu.SemaphoreType.DMA,
            pltpu.SemaphoreType.DMA,
        ],
        compiler_params=pltpu.CompilerParams(use_tc_tiling_on_sc=False),
    )
    def agg_kernel(hp, row3, col3, zblk, out,
                   row_v, col_v, bufa, bufb, acc, sema, semb):
        c = lax.axis_index("c")
        s = lax.axis_index("s")
        w = s * NC + c
        off = pl.multiple_of(jnp.minimum(s * RPT, N - RPT), 8)
        pltpu.sync_copy(row3.at[w], row_v)
        pltpu.sync_copy(col3.at[w], col_v)
        pltpu.sync_copy(zblk, acc.at[pl.ds(off, RPT)])
        plsc.subcore_barrier()

        pltpu.async_copy(hp.at[row_v.at[0]], bufa, sema)
        pltpu.async_copy(hp.at[row_v.at[1]], bufb, semb)

        def body(i, carry):
            j0 = 2 * i
            pltpu.make_async_copy(hp.at[row_v.at[0]], bufa, sema).wait()
            pltpu.sync_copy(bufa, acc.at[col_v.at[j0]], add=True)

            @pl.when(i < NB // 2 - 1)
            def _():
                pltpu.async_copy(hp.at[row_v.at[j0 + 2]], bufa, sema)

            pltpu.make_async_copy(hp.at[row_v.at[0]], bufb, semb).wait()
            pltpu.sync_copy(bufb, acc.at[col_v.at[j0 + 1]], add=True)

            @pl.when(i < NB // 2 - 1)
            def _():
                pltpu.async_copy(hp.at[row_v.at[j0 + 3]], bufb, semb)

            return carry

        lax.fori_loop(0, NB // 2, body, 0)
        plsc.subcore_barrier()
        pltpu.sync_copy(acc.at[pl.ds(off, RPT)], out.at[c, pl.ds(off, RPT)])

    return agg_kernel


_deg_kernel = _make_deg_kernel()
_agg_kernel = _make_agg_kernel(DH2)


BN = 400


def _scale_mm_body(d0_ref, d1_ref, x_ref, w_ref, lo_ref, hi_ref):
    dis = lax.rsqrt(d0_ref[...] + d1_ref[...] + 1.0)
    h = dis * jnp.dot(x_ref[...], w_ref[...],
                      preferred_element_type=jnp.float32)
    lo_ref[...] = h[:, :DH2]
    hi_ref[...] = h[:, DH2:]


def _combine_mm_body(d0_ref, d1_ref, al0_ref, al1_ref, ah0_ref, ah1_ref,
                     lo_ref, hi_ref, b_ref, w_ref, *out_refs):
    dis = lax.rsqrt(d0_ref[...] + d1_ref[...] + 1.0)
    agg_lo = al0_ref[0] + al1_ref[0] + lo_ref[...]
    agg_hi = ah0_ref[0] + ah1_ref[0] + hi_ref[...]
    t = dis * jnp.concatenate([agg_lo, agg_hi], axis=1) + b_ref[...]
    t = jnp.maximum(t, 0.0)
    h = dis * jnp.dot(t, w_ref[...], preferred_element_type=jnp.float32)
    if len(out_refs) == 2:
        out_refs[0][...] = h[:, :DH2]
        out_refs[1][...] = h[:, DH2:]
    else:
        out_refs[0][...] = h


def _final_body(d0_ref, d1_ref, a0_ref, a1_ref, hp_ref, b_ref, o_ref):
    dis = lax.rsqrt(d0_ref[...] + d1_ref[...] + 1.0)
    o_ref[...] = dis * (a0_ref[0] + a1_ref[0] + hp_ref[...]) + b_ref[...]


def _deg_specs():
    return [
        pl.BlockSpec((BN, 1), lambda i: (i, 0)),
        pl.BlockSpec((BN, 1), lambda i: (i, 0)),
    ]


def _agg_pair_specs():
    return [
        pl.BlockSpec((1, BN, DH2), lambda i: (0, i, 0)),
        pl.BlockSpec((1, BN, DH2), lambda i: (1, i, 0)),
    ]


def _half_out_specs():
    return [
        pl.BlockSpec((BN, DH2), lambda i: (i, 0)),
        pl.BlockSpec((BN, DH2), lambda i: (i, 0)),
    ]


def _half_out_shapes():
    return [
        jax.ShapeDtypeStruct((N, DH2), jnp.float32),
        jax.ShapeDtypeStruct((N, DH2), jnp.float32),
    ]


def _scale_mm(d0, d1, x, w):
    return pl.pallas_call(
        _scale_mm_body,
        grid=(N // BN,),
        in_specs=_deg_specs() + [
            pl.BlockSpec((BN, x.shape[1]), lambda i: (i, 0)),
            pl.BlockSpec(w.shape, lambda i: (0, 0)),
        ],
        out_specs=_half_out_specs(),
        out_shape=_half_out_shapes(),
    )(d0, d1, x, w)


def _combine_mm(d0, d1, agg_lo, agg_hi, hp_lo, hp_hi, b, w, split_out):
    d_out = w.shape[1]
    if split_out:
        out_specs, out_shape = _half_out_specs(), _half_out_shapes()
    else:
        out_specs = pl.BlockSpec((BN, d_out), lambda i: (i, 0))
        out_shape = jax.ShapeDtypeStruct((N, d_out), jnp.float32)
    return pl.pallas_call(
        _combine_mm_body,
        grid=(N // BN,),
        in_specs=_deg_specs() + _agg_pair_specs() + _agg_pair_specs() + [
            pl.BlockSpec((BN, DH2), lambda i: (i, 0)),
            pl.BlockSpec((BN, DH2), lambda i: (i, 0)),
            pl.BlockSpec((1, D_H), lambda i: (0, 0)),
            pl.BlockSpec(w.shape, lambda i: (0, 0)),
        ],
        out_specs=out_specs,
        out_shape=out_shape,
    )(d0, d1, agg_lo, agg_lo, agg_hi, agg_hi, hp_lo, hp_hi, b, w)


def _final(d0, d1, agg, hp, b):
    d = hp.shape[1]
    return pl.pallas_call(
        _final_body,
        grid=(N // BN,),
        in_specs=_deg_specs() + [
            pl.BlockSpec((1, BN, d), lambda i: (0, i, 0)),
            pl.BlockSpec((1, BN, d), lambda i: (1, i, 0)),
            pl.BlockSpec((BN, d), lambda i: (i, 0)),
            pl.BlockSpec((1, d), lambda i: (0, 0)),
        ],
        out_specs=pl.BlockSpec((BN, d), lambda i: (i, 0)),
        out_shape=jax.ShapeDtypeStruct((N, d), jnp.float32),
    )(d0, d1, agg, agg, hp, b)


def kernel(x, edge_index, W1, b1, W2, b2, W3, b3):
    row3 = edge_index[0].reshape(NW, NB, K)
    col3 = edge_index[1].reshape(NW, NB, K)
    z16 = jnp.zeros((RPT, DEGW), jnp.float32)
    z64 = jnp.zeros((RPT, DH2), jnp.float32)

    degp = _deg_kernel(col3, z16)
    d0 = degp[0, :, :1]
    d1 = degp[1, :, :1]

    h1_lo, h1_hi = _scale_mm(d0, d1, x, W1)
    a1_lo = _agg_kernel(h1_lo, row3, col3, z64)
    a1_hi = _agg_kernel(h1_hi, row3, col3, z64)
    h2_lo, h2_hi = _combine_mm(d0, d1, a1_lo, a1_hi, h1_lo, h1_hi,
                               b1.reshape(1, -1), W2, split_out=True)
    a2_lo = _agg_kernel(h2_lo, row3, col3, z64)
    a2_hi = _agg_kernel(h2_hi, row3, col3, z64)
    h3p = _combine_mm(d0, d1, a2_lo, a2_hi, h2_lo, h2_hi,
                      b2.reshape(1, -1), W3, split_out=False)
    a3 = _agg_kernel(h3p, row3, col3, z64)
    return _final(d0, d1, a3, h3p, b3.reshape(1, -1))


# speedup vs baseline: 22.7161x; 1.0699x over previous
import functools

import jax
import jax.numpy as jnp
from jax import lax
from jax.experimental import pallas as pl
from jax.experimental.pallas import tpu as pltpu
from jax.experimental.pallas import tpu_sc as plsc

N = 10000
E = 320000
D_IN = 128
D_H = 128
D_OUT = 64
DH2 = D_H // 2

NC = 2
NS = 16
NW = NC * NS
K = 100
NB = E // (NW * K)
RPT = 632
DEGW = 16

_MESH = plsc.VectorSubcoreMesh(
    core_axis_name="c", subcore_axis_name="s", num_cores=NC, num_subcores=NS
)


def _make_deg_kernel():
    @functools.partial(
        pl.kernel,
        out_type=jax.ShapeDtypeStruct((NC, N, DEGW), jnp.float32),
        mesh=_MESH,
        scratch_types=[
            pltpu.VMEM((NB, K), jnp.int32),
            pltpu.VMEM((K, DEGW), jnp.float32),
            pltpu.VMEM_SHARED((N, DEGW), jnp.float32),
            pltpu.SemaphoreType.DMA,
        ],
        compiler_params=pltpu.CompilerParams(use_tc_tiling_on_sc=False),
    )
    def deg_kernel(col3, zblk, out, col_v, ones_v, acc, sem):
        c = lax.axis_index("c")
        s = lax.axis_index("s")
        w = s * NC + c
        off = pl.multiple_of(jnp.minimum(s * RPT, N - RPT), 8)
        pltpu.sync_copy(col3.at[w], col_v)

        def fill(r, carry):
            ones_v[r, :] = jnp.ones((DEGW,), jnp.float32)
            return carry

        lax.fori_loop(0, K, fill, 0)
        pltpu.sync_copy(zblk, acc.at[pl.ds(off, RPT)])
        plsc.subcore_barrier()

        W = 8

        def body(j, carry):
            @pl.when(j >= W)
            def _():
                pltpu.make_async_copy(ones_v, acc.at[col_v.at[0]], sem).wait()

            pltpu.async_copy(ones_v, acc.at[col_v.at[j]], sem, add=True)
            return carry

        lax.fori_loop(0, NB, body, 0)

        def drain(j, carry):
            pltpu.make_async_copy(ones_v, acc.at[col_v.at[0]], sem).wait()
            return carry

        lax.fori_loop(0, W, drain, 0)
        plsc.subcore_barrier()
        pltpu.sync_copy(acc.at[pl.ds(off, RPT)], out.at[c, pl.ds(off, RPT)])

    return deg_kernel


def _make_agg_kernel(D):
    @functools.partial(
        pl.kernel,
        out_type=jax.ShapeDtypeStruct((NC, N, D), jnp.float32),
        mesh=_MESH,
        scratch_types=[
            pltpu.VMEM((NB, K), jnp.int32),
            pltpu.VMEM((NB, K), jnp.int32),
            [pltpu.VMEM((K, D), jnp.float32) for _ in range(4)],
            [pltpu.SemaphoreType.DMA for _ in range(4)],
            [pltpu.SemaphoreType.DMA for _ in range(4)],
            pltpu.VMEM_SHARED((N, D), jnp.float32),
        ],
        compiler_params=pltpu.CompilerParams(use_tc_tiling_on_sc=False),
    )
    def agg_kernel(hp, row3, col3, zblk, out,
                   row_v, col_v, bufs, gsems, ssems, acc):
        c = lax.axis_index("c")
        s = lax.axis_index("s")
        w = s * NC + c
        off = pl.multiple_of(jnp.minimum(s * RPT, N - RPT), 8)
        pltpu.sync_copy(row3.at[w], row_v)
        pltpu.sync_copy(col3.at[w], col_v)
        pltpu.sync_copy(zblk, acc.at[pl.ds(off, RPT)])
        plsc.subcore_barrier()

        pltpu.async_copy(hp.at[row_v.at[0]], bufs[0], gsems[0])
        pltpu.async_copy(hp.at[row_v.at[1]], bufs[1], gsems[1])

        def group(i, carry):
            for u in range(4):
                j = 4 * i + u
                buf, gsem, ssem = bufs[u], gsems[u], ssems[u]
                pltpu.make_async_copy(hp.at[row_v.at[0]], buf, gsem).wait()
                pltpu.async_copy(buf, acc.at[col_v.at[j]], ssem, add=True)

                u2 = (u + 2) % 4
                buf2, gsem2, ssem2 = bufs[u2], gsems[u2], ssems[u2]

                @pl.when(j >= 2)
                def _():
                    pltpu.make_async_copy(
                        buf2, acc.at[col_v.at[0]], ssem2).wait()

                @pl.when(j + 2 < NB)
                def _():
                    pltpu.async_copy(hp.at[row_v.at[j + 2]], buf2, gsem2)

            return carry

        lax.fori_loop(0, NB // 4, group, 0)
        pltpu.make_async_copy(bufs[2], acc.at[col_v.at[0]], ssems[2]).wait()
        pltpu.make_async_copy(bufs[3], acc.at[col_v.at[0]], ssems[3]).wait()
        plsc.subcore_barrier()
        pltpu.sync_copy(acc.at[pl.ds(off, RPT)], out.at[c, pl.ds(off, RPT)])

    return agg_kernel


_deg_kernel = _make_deg_kernel()
_agg_kernel = _make_agg_kernel(DH2)


BN = 400


def _scale_mm_body(d0_ref, d1_ref, x_ref, w_ref, lo_ref, hi_ref):
    dis = lax.rsqrt(d0_ref[...] + d1_ref[...] + 1.0)
    h = dis * jnp.dot(x_ref[...], w_ref[...],
                      preferred_element_type=jnp.float32)
    lo_ref[...] = h[:, :DH2]
    hi_ref[...] = h[:, DH2:]


def _combine_mm_body(d0_ref, d1_ref, al0_ref, al1_ref, ah0_ref, ah1_ref,
                     lo_ref, hi_ref, b_ref, w_ref, *out_refs):
    dis = lax.rsqrt(d0_ref[...] + d1_ref[...] + 1.0)
    agg_lo = al0_ref[0] + al1_ref[0] + lo_ref[...]
    agg_hi = ah0_ref[0] + ah1_ref[0] + hi_ref[...]
    t = dis * jnp.concatenate([agg_lo, agg_hi], axis=1) + b_ref[...]
    t = jnp.maximum(t, 0.0)
    h = dis * jnp.dot(t, w_ref[...], preferred_element_type=jnp.float32)
    if len(out_refs) == 2:
        out_refs[0][...] = h[:, :DH2]
        out_refs[1][...] = h[:, DH2:]
    else:
        out_refs[0][...] = h


def _final_body(d0_ref, d1_ref, a0_ref, a1_ref, hp_ref, b_ref, o_ref):
    dis = lax.rsqrt(d0_ref[...] + d1_ref[...] + 1.0)
    o_ref[...] = dis * (a0_ref[0] + a1_ref[0] + hp_ref[...]) + b_ref[...]


def _deg_specs():
    return [
        pl.BlockSpec((BN, 1), lambda i: (i, 0)),
        pl.BlockSpec((BN, 1), lambda i: (i, 0)),
    ]


def _agg_pair_specs():
    return [
        pl.BlockSpec((1, BN, DH2), lambda i: (0, i, 0)),
        pl.BlockSpec((1, BN, DH2), lambda i: (1, i, 0)),
    ]


def _half_out_specs():
    return [
        pl.BlockSpec((BN, DH2), lambda i: (i, 0)),
        pl.BlockSpec((BN, DH2), lambda i: (i, 0)),
    ]


def _half_out_shapes():
    return [
        jax.ShapeDtypeStruct((N, DH2), jnp.float32),
        jax.ShapeDtypeStruct((N, DH2), jnp.float32),
    ]


def _scale_mm(d0, d1, x, w):
    return pl.pallas_call(
        _scale_mm_body,
        grid=(N // BN,),
        in_specs=_deg_specs() + [
            pl.BlockSpec((BN, x.shape[1]), lambda i: (i, 0)),
            pl.BlockSpec(w.shape, lambda i: (0, 0)),
        ],
        out_specs=_half_out_specs(),
        out_shape=_half_out_shapes(),
    )(d0, d1, x, w)


def _combine_mm(d0, d1, agg_lo, agg_hi, hp_lo, hp_hi, b, w, split_out):
    d_out = w.shape[1]
    if split_out:
        out_specs, out_shape = _half_out_specs(), _half_out_shapes()
    else:
        out_specs = pl.BlockSpec((BN, d_out), lambda i: (i, 0))
        out_shape = jax.ShapeDtypeStruct((N, d_out), jnp.float32)
    return pl.pallas_call(
        _combine_mm_body,
        grid=(N // BN,),
        in_specs=_deg_specs() + _agg_pair_specs() + _agg_pair_specs() + [
            pl.BlockSpec((BN, DH2), lambda i: (i, 0)),
            pl.BlockSpec((BN, DH2), lambda i: (i, 0)),
            pl.BlockSpec((1, D_H), lambda i: (0, 0)),
            pl.BlockSpec(w.shape, lambda i: (0, 0)),
        ],
        out_specs=out_specs,
        out_shape=out_shape,
    )(d0, d1, agg_lo, agg_lo, agg_hi, agg_hi, hp_lo, hp_hi, b, w)


def _final(d0, d1, agg, hp, b):
    d = hp.shape[1]
    return pl.pallas_call(
        _final_body,
        grid=(N // BN,),
        in_specs=_deg_specs() + [
            pl.BlockSpec((1, BN, d), lambda i: (0, i, 0)),
            pl.BlockSpec((1, BN, d), lambda i: (1, i, 0)),
            pl.BlockSpec((BN, d), lambda i: (i, 0)),
            pl.BlockSpec((1, d), lambda i: (0, 0)),
        ],
        out_specs=pl.BlockSpec((BN, d), lambda i: (i, 0)),
        out_shape=jax.ShapeDtypeStruct((N, d), jnp.float32),
    )(d0, d1, agg, agg, hp, b)


def kernel(x, edge_index, W1, b1, W2, b2, W3, b3):
    row3 = edge_index[0].reshape(NW, NB, K)
    col3 = edge_index[1].reshape(NW, NB, K)
    z16 = jnp.zeros((RPT, DEGW), jnp.float32)
    z64 = jnp.zeros((RPT, DH2), jnp.float32)

    degp = _deg_kernel(col3, z16)
    d0 = degp[0, :, :1]
    d1 = degp[1, :, :1]

    h1_lo, h1_hi = _scale_mm(d0, d1, x, W1)
    a1_lo = _agg_kernel(h1_lo, row3, col3, z64)
    a1_hi = _agg_kernel(h1_hi, row3, col3, z64)
    h2_lo, h2_hi = _combine_mm(d0, d1, a1_lo, a1_hi, h1_lo, h1_hi,
                               b1.reshape(1, -1), W2, split_out=True)
    a2_lo = _agg_kernel(h2_lo, row3, col3, z64)
    a2_hi = _agg_kernel(h2_hi, row3, col3, z64)
    h3p = _combine_mm(d0, d1, a2_lo, a2_hi, h2_lo, h2_hi,
                      b2.reshape(1, -1), W3, split_out=False)
    a3 = _agg_kernel(h3p, row3, col3, z64)
    return _final(d0, d1, a3, h3p, b3.reshape(1, -1))


# speedup vs baseline: 24.2252x; 1.0664x over previous
import functools

import jax
import jax.numpy as jnp
from jax import lax
from jax.experimental import pallas as pl
from jax.experimental.pallas import tpu as pltpu
from jax.experimental.pallas import tpu_sc as plsc

N = 10000
E = 320000
D_IN = 128
D_H = 128
D_OUT = 64
DH2 = D_H // 2

NC = 2
NS = 16
NW = NC * NS
K = 125
NB = E // (NW * K)
RPT = 632
DEGW = 16

_MESH = plsc.VectorSubcoreMesh(
    core_axis_name="c", subcore_axis_name="s", num_cores=NC, num_subcores=NS
)


def _make_deg_kernel():
    @functools.partial(
        pl.kernel,
        out_type=jax.ShapeDtypeStruct((NC, N, DEGW), jnp.float32),
        mesh=_MESH,
        scratch_types=[
            pltpu.VMEM((NB, K), jnp.int32),
            pltpu.VMEM((K, DEGW), jnp.float32),
            pltpu.VMEM_SHARED((N, DEGW), jnp.float32),
            pltpu.SemaphoreType.DMA,
        ],
        compiler_params=pltpu.CompilerParams(use_tc_tiling_on_sc=False),
    )
    def deg_kernel(ei4, zblk, out, col_v, ones_v, acc, sem):
        c = lax.axis_index("c")
        s = lax.axis_index("s")
        w = s * NC + c
        off = pl.multiple_of(jnp.minimum(s * RPT, N - RPT), 8)
        pltpu.sync_copy(ei4.at[1, w], col_v)

        def fill(r, carry):
            ones_v[r, :] = jnp.ones((DEGW,), jnp.float32)
            return carry

        lax.fori_loop(0, K, fill, 0)
        pltpu.sync_copy(zblk, acc.at[pl.ds(off, RPT)])
        plsc.subcore_barrier()

        W = 8

        def body(j, carry):
            @pl.when(j >= W)
            def _():
                pltpu.make_async_copy(ones_v, acc.at[col_v.at[0]], sem).wait()

            pltpu.async_copy(ones_v, acc.at[col_v.at[j]], sem, add=True)
            return carry

        lax.fori_loop(0, NB, body, 0)

        def drain(j, carry):
            pltpu.make_async_copy(ones_v, acc.at[col_v.at[0]], sem).wait()
            return carry

        lax.fori_loop(0, W, drain, 0)
        plsc.subcore_barrier()
        pltpu.sync_copy(acc.at[pl.ds(off, RPT)], out.at[c, pl.ds(off, RPT)])

    return deg_kernel


def _make_agg_kernel(nphase):
    D = DH2
    out_one = jax.ShapeDtypeStruct((NC, N, D), jnp.float32)

    @functools.partial(
        pl.kernel,
        out_type=[out_one] * nphase,
        mesh=_MESH,
        scratch_types=[
            pltpu.VMEM((NB, K), jnp.int32),
            pltpu.VMEM((NB, K), jnp.int32),
            [pltpu.VMEM((K, D), jnp.float32) for _ in range(4)],
            [pltpu.SemaphoreType.DMA for _ in range(4)],
            [pltpu.SemaphoreType.DMA for _ in range(4)],
            pltpu.VMEM_SHARED((N, D), jnp.float32),
        ],
        compiler_params=pltpu.CompilerParams(use_tc_tiling_on_sc=False),
    )
    def agg_kernel(*refs):
        hps = refs[:nphase]
        ei4, zblk = refs[nphase], refs[nphase + 1]
        outs = refs[nphase + 2:2 * nphase + 2]
        row_v, col_v, bufs, gsems, ssems, acc = refs[2 * nphase + 2:]
        c = lax.axis_index("c")
        s = lax.axis_index("s")
        w = s * NC + c
        off = pl.multiple_of(jnp.minimum(s * RPT, N - RPT), 8)
        pltpu.sync_copy(ei4.at[0, w], row_v)
        pltpu.sync_copy(ei4.at[1, w], col_v)
        pltpu.sync_copy(zblk, acc.at[pl.ds(off, RPT)])
        plsc.subcore_barrier()

        def run_phase(hp, out, last):
            pltpu.async_copy(hp.at[row_v.at[0]], bufs[0], gsems[0])
            pltpu.async_copy(hp.at[row_v.at[1]], bufs[1], gsems[1])

            def group(i, carry):
                for u in range(4):
                    j = 4 * i + u
                    buf, gsem, ssem = bufs[u], gsems[u], ssems[u]
                    pltpu.make_async_copy(
                        hp.at[row_v.at[0]], buf, gsem).wait()
                    pltpu.async_copy(buf, acc.at[col_v.at[j]], ssem,
                                     add=True)

                    u2 = (u + 2) % 4
                    buf2, gsem2, ssem2 = bufs[u2], gsems[u2], ssems[u2]

                    @pl.when(j >= 2)
                    def _():
                        pltpu.make_async_copy(
                            buf2, acc.at[col_v.at[0]], ssem2).wait()

                    @pl.when(j + 2 < NB)
                    def _():
                        pltpu.async_copy(hp.at[row_v.at[j + 2]], buf2, gsem2)

                return carry

            lax.fori_loop(0, NB // 4, group, 0)
            pltpu.make_async_copy(
                bufs[(NB - 2) % 4], acc.at[col_v.at[0]],
                ssems[(NB - 2) % 4]).wait()
            pltpu.make_async_copy(
                bufs[(NB - 1) % 4], acc.at[col_v.at[0]],
                ssems[(NB - 1) % 4]).wait()
            plsc.subcore_barrier()
            pltpu.sync_copy(acc.at[pl.ds(off, RPT)],
                            out.at[c, pl.ds(off, RPT)])
            if not last:
                pltpu.sync_copy(zblk, acc.at[pl.ds(off, RPT)])
                plsc.subcore_barrier()

        for p in range(nphase):
            run_phase(hps[p], outs[p], p == nphase - 1)

    return agg_kernel


_deg_kernel = _make_deg_kernel()
_agg_kernel2 = _make_agg_kernel(2)
_agg_kernel1 = _make_agg_kernel(1)


BN = 400


def _dis(d0_ref, d1_ref):
    return lax.rsqrt(d0_ref[0][:, :1] + d1_ref[0][:, :1] + 1.0)


def _scale_mm_body(d0_ref, d1_ref, x_ref, w_ref, lo_ref, hi_ref):
    dis = _dis(d0_ref, d1_ref)
    h = dis * jnp.dot(x_ref[...], w_ref[...],
                      preferred_element_type=jnp.float32)
    lo_ref[...] = h[:, :DH2]
    hi_ref[...] = h[:, DH2:]


def _combine_mm_body(d0_ref, d1_ref, al0_ref, al1_ref, ah0_ref, ah1_ref,
                     lo_ref, hi_ref, b_ref, w_ref, *out_refs):
    dis = _dis(d0_ref, d1_ref)
    agg_lo = al0_ref[0] + al1_ref[0] + lo_ref[...]
    agg_hi = ah0_ref[0] + ah1_ref[0] + hi_ref[...]
    t = dis * jnp.concatenate([agg_lo, agg_hi], axis=1) + b_ref[...]
    t = jnp.maximum(t, 0.0)
    h = dis * jnp.dot(t, w_ref[...], preferred_element_type=jnp.float32)
    if len(out_refs) == 2:
        out_refs[0][...] = h[:, :DH2]
        out_refs[1][...] = h[:, DH2:]
    else:
        out_refs[0][...] = h


def _final_body(d0_ref, d1_ref, a0_ref, a1_ref, hp_ref, b_ref, o_ref):
    dis = _dis(d0_ref, d1_ref)
    o_ref[...] = dis * (a0_ref[0] + a1_ref[0] + hp_ref[...]) + b_ref[...]


def _deg_specs():
    return [
        pl.BlockSpec((1, BN, DEGW), lambda i: (0, i, 0)),
        pl.BlockSpec((1, BN, DEGW), lambda i: (1, i, 0)),
    ]


def _agg_pair_specs():
    return [
        pl.BlockSpec((1, BN, DH2), lambda i: (0, i, 0)),
        pl.BlockSpec((1, BN, DH2), lambda i: (1, i, 0)),
    ]


def _half_out_specs():
    return [
        pl.BlockSpec((BN, DH2), lambda i: (i, 0)),
        pl.BlockSpec((BN, DH2), lambda i: (i, 0)),
    ]


def _half_out_shapes():
    return [
        jax.ShapeDtypeStruct((N, DH2), jnp.float32),
        jax.ShapeDtypeStruct((N, DH2), jnp.float32),
    ]


def _scale_mm(degp, x, w):
    return pl.pallas_call(
        _scale_mm_body,
        grid=(N // BN,),
        in_specs=_deg_specs() + [
            pl.BlockSpec((BN, x.shape[1]), lambda i: (i, 0)),
            pl.BlockSpec(w.shape, lambda i: (0, 0)),
        ],
        out_specs=_half_out_specs(),
        out_shape=_half_out_shapes(),
    )(degp, degp, x, w)


def _combine_mm(degp, agg_lo, agg_hi, hp_lo, hp_hi, b, w, split_out):
    d_out = w.shape[1]
    if split_out:
        out_specs, out_shape = _half_out_specs(), _half_out_shapes()
    else:
        out_specs = pl.BlockSpec((BN, d_out), lambda i: (i, 0))
        out_shape = jax.ShapeDtypeStruct((N, d_out), jnp.float32)
    return pl.pallas_call(
        _combine_mm_body,
        grid=(N // BN,),
        in_specs=_deg_specs() + _agg_pair_specs() + _agg_pair_specs() + [
            pl.BlockSpec((BN, DH2), lambda i: (i, 0)),
            pl.BlockSpec((BN, DH2), lambda i: (i, 0)),
            pl.BlockSpec((1, D_H), lambda i: (0, 0)),
            pl.BlockSpec(w.shape, lambda i: (0, 0)),
        ],
        out_specs=out_specs,
        out_shape=out_shape,
    )(degp, degp, agg_lo, agg_lo, agg_hi, agg_hi, hp_lo, hp_hi, b, w)


def _final(degp, agg, hp, b):
    d = hp.shape[1]
    return pl.pallas_call(
        _final_body,
        grid=(N // BN,),
        in_specs=_deg_specs() + [
            pl.BlockSpec((1, BN, d), lambda i: (0, i, 0)),
            pl.BlockSpec((1, BN, d), lambda i: (1, i, 0)),
            pl.BlockSpec((BN, d), lambda i: (i, 0)),
            pl.BlockSpec((1, d), lambda i: (0, 0)),
        ],
        out_specs=pl.BlockSpec((BN, d), lambda i: (i, 0)),
        out_shape=jax.ShapeDtypeStruct((N, d), jnp.float32),
    )(degp, degp, agg, agg, hp, b)


def kernel(x, edge_index, W1, b1, W2, b2, W3, b3):
    ei4 = edge_index.reshape(2, NW, NB, K)
    z16 = jnp.zeros((RPT, DEGW), jnp.float32)
    z64 = jnp.zeros((RPT, DH2), jnp.float32)

    degp = _deg_kernel(ei4, z16)

    h1_lo, h1_hi = _scale_mm(degp, x, W1)
    a1_lo, a1_hi = _agg_kernel2(h1_lo, h1_hi, ei4, z64)
    h2_lo, h2_hi = _combine_mm(degp, a1_lo, a1_hi, h1_lo, h1_hi,
                               b1.reshape(1, -1), W2, split_out=True)
    a2_lo, a2_hi = _agg_kernel2(h2_lo, h2_hi, ei4, z64)
    h3p = _combine_mm(degp, a2_lo, a2_hi, h2_lo, h2_hi,
                      b2.reshape(1, -1), W3, split_out=False)
    (a3,) = _agg_kernel1(h3p, ei4, z64)
    return _final(degp, a3, h3p, b3.reshape(1, -1))
